# baseline (speedup 1.0000x reference)
import jax
import jax.numpy as jnp
from jax import lax
from jax.experimental import pallas as pl
from jax.experimental.pallas import tpu as pltpu
from jax.experimental.pallas import tpu_sc as plsc

B = 4096
S = 200
D = 64
H = 256
C = 2
V = 1000000

NC = 2
NS = 16
NW = NC * NS
BPW = B // NW

_S0 = 128
_S1 = S - _S0

W = 256
NBLK = V // W
VTAIL = V - NBLK * W


def _sc_fmt_body(embT_hbm, out_hbm, in0, in1, ob0, ob1, tin, tout,
                 si0, si1, so0, so1):
    wid = lax.axis_index("s") * NC + lax.axis_index("c")

    cvecs = [lax.iota(jnp.int32, 16) + 16 * m for m in range(4)]

    def fire_in(blk, buf, sem):
        pltpu.async_copy(embT_hbm.at[pl.ds(0, D), pl.ds(W * blk, W)],
                         buf.at[pl.ds(0, D), pl.ds(0, W)], sem)

    def wait_in(buf, sem):
        pltpu.make_async_copy(embT_hbm.at[pl.ds(0, D), pl.ds(0, W)],
                              buf.at[pl.ds(0, D), pl.ds(0, W)], sem).wait()

    def fire_out(blk, buf, sem):
        pltpu.async_copy(buf, out_hbm.at[pl.ds((W // 2) * blk, W // 2)], sem)

    def wait_out(buf, sem):
        pltpu.make_async_copy(buf, out_hbm.at[pl.ds(0, W // 2)], sem).wait()

    iot = lax.iota(jnp.int32, 16)
    qvs = [(16 * m + iot) >> 1 for m in range(W // 16)]
    jbs = [((16 * m + iot) & 1) << 6 for m in range(W // 16)]

    def transpose_block(src, dst):
        @plsc.parallel_loop(0, D, unroll=4)
        def cbody(c):
            cv = jnp.full((16,), c, jnp.int32)
            for m in range(W // 16):
                v = src[c, pl.ds(16 * m, 16)]
                plsc.store_scatter(dst, [qvs[m], jbs[m] + cv], v)

    n = (NBLK - wid + NW - 1) // NW

    def blkof(k):
        return wid + NW * k

    fire_in(blkof(0), in0, si0)

    @pl.when(n >= 2)
    def _():
        fire_in(blkof(1), in1, si1)

    def loop_body(k, carry):
        def stage(buf_in, buf_out, sem_in, sem_out):
            wait_in(buf_in, sem_in)

            @pl.when(k >= 2)
            def _():
                wait_out(buf_out, sem_out)

            transpose_block(buf_in, buf_out)
            fire_out(blkof(k), buf_out, sem_out)

            @pl.when(k + 2 < n)
            def _():
                fire_in(blkof(k + 2), buf_in, sem_in)

        @pl.when(k % 2 == 0)
        def _():
            stage(in0, ob0, si0, so0)

        @pl.when(k % 2 == 1)
        def _():
            stage(in1, ob1, si1, so1)

        return carry

    lax.fori_loop(0, n, loop_body, jnp.int32(0))

    @pl.when(n >= 1)
    def _():
        wait_out(ob0, so0)

    @pl.when(n >= 2)
    def _():
        wait_out(ob1, so1)

    @pl.when(wid == NW - 1)
    def _():
        pltpu.sync_copy(embT_hbm.at[pl.ds(0, D), pl.ds(W * NBLK, VTAIL)],
                        tin)

        def qbody(q, carry):
            s0v = jnp.full((16,), 2 * q, jnp.int32)
            s1v = s0v + 1
            for m in range(4):
                tout[q, pl.ds(16 * m, 16)] = plsc.load_gather(tin, [cvecs[m], s0v])
            for m in range(4):
                tout[q, pl.ds(64 + 16 * m, 16)] = plsc.load_gather(tin, [cvecs[m], s1v])
            return carry

        lax.fori_loop(0, VTAIL // 2, qbody, jnp.int32(0))
        pltpu.sync_copy(tout, out_hbm.at[pl.ds((W // 2) * NBLK, VTAIL // 2)])


@jax.jit
def _sc_fmt(embT):
    mesh = plsc.VectorSubcoreMesh(core_axis_name="c", subcore_axis_name="s")
    return pl.kernel(
        _sc_fmt_body,
        out_type=jax.ShapeDtypeStruct((V // 2, 128), jnp.float32),
        mesh=mesh,
        scratch_types=[
            pltpu.VMEM((D, W + 1), jnp.float32),
            pltpu.VMEM((D, W + 1), jnp.float32),
            pltpu.VMEM((W // 2, 128), jnp.float32),
            pltpu.VMEM((W // 2, 128), jnp.float32),
            pltpu.VMEM((D, VTAIL), jnp.float32),
            pltpu.VMEM((VTAIL // 2, 128), jnp.float32),
            pltpu.SemaphoreType.DMA,
            pltpu.SemaphoreType.DMA,
            pltpu.SemaphoreType.DMA,
            pltpu.SemaphoreType.DMA,
        ],
        compiler_params=pltpu.CompilerParams(use_tc_tiling_on_sc=True,
                                             needs_layout_passes=False),
    )(embT)


def _sc_pool_body(x_hbm, emb_hbm, out_hbm, idx_v, rows0, rows1, out_v, sem0, sem1):
    wid = lax.axis_index("s") * NC + lax.axis_index("c")
    base = wid * BPW

    pltpu.sync_copy(x_hbm.at[pl.ds(base, BPW)], idx_v)

    def fire(b, buf, sem):
        pltpu.async_copy(emb_hbm.at[idx_v.at[b, pl.ds(0, _S0)]],
                         buf.at[pl.ds(0, _S0)], sem)
        pltpu.async_copy(emb_hbm.at[idx_v.at[b, pl.ds(_S0, _S1)]],
                         buf.at[pl.ds(_S0, _S1)], sem)

    def drain(b, buf, sem):
        pltpu.make_async_copy(emb_hbm.at[idx_v.at[b, pl.ds(0, _S0)]],
                              buf.at[pl.ds(0, _S0)], sem).wait()
        pltpu.make_async_copy(emb_hbm.at[idx_v.at[b, pl.ds(_S0, _S1)]],
                              buf.at[pl.ds(_S0, _S1)], sem).wait()

    def process(b, buf, sem):
        drain(b, buf, sem)
        zero = jnp.zeros((16,), jnp.float32)

        def acc_body(j, accs):
            return tuple(a + buf[j, pl.ds(16 * k, 16)] for k, a in enumerate(accs))

        a0, a1, a2, a3 = lax.fori_loop(0, S, acc_body, (zero, zero, zero, zero))

        out_v[b, pl.ds(0, 16)] = a0
        out_v[b, pl.ds(16, 16)] = a1
        out_v[b, pl.ds(32, 16)] = a2
        out_v[b, pl.ds(48, 16)] = a3

    fire(0, rows0, sem0)
    fire(1, rows1, sem1)

    def loop_body(g, carry):
        b = 2 * g
        process(b, rows0, sem0)

        @pl.when(b + 2 < BPW)
        def _():
            fire(b + 2, rows0, sem0)

        process(b + 1, rows1, sem1)

        @pl.when(b + 3 < BPW)
        def _():
            fire(b + 3, rows1, sem1)

        return carry

    lax.fori_loop(0, BPW // 2, loop_body, jnp.int32(0))

    pltpu.sync_copy(out_v, out_hbm.at[pl.ds(base, BPW)])


@jax.jit
def _sc_pool(x, table):
    mesh = plsc.VectorSubcoreMesh(core_axis_name="c", subcore_axis_name="s")
    return pl.kernel(
        _sc_pool_body,
        out_type=jax.ShapeDtypeStruct((B, D), jnp.float32),
        mesh=mesh,
        scratch_types=[
            pltpu.VMEM((BPW, S), jnp.int32),
            pltpu.VMEM((S, D), jnp.float32),
            pltpu.VMEM((S, D), jnp.float32),
            pltpu.VMEM((BPW, D), jnp.float32),
            pltpu.SemaphoreType.DMA,
            pltpu.SemaphoreType.DMA,
        ],
        compiler_params=pltpu.CompilerParams(use_tc_tiling_on_sc=False),
    )(x, table)


def _mlp_body(x_ref, summed_ref, W1_ref, b1_ref, W2_ref, b2_ref, out_ref):
    denom = jnp.sum((x_ref[...] != 0).astype(jnp.float32), axis=1, keepdims=True)
    avg = summed_ref[...] / jnp.maximum(denom, 1.0)
    h = jnp.dot(avg, W1_ref[...], preferred_element_type=jnp.float32)
    h = jnp.maximum(h + b1_ref[...], 0.0)
    logits = jnp.dot(h, W2_ref[...], preferred_element_type=jnp.float32)
    logits = logits + b2_ref[...]
    m = jnp.max(logits, axis=1, keepdims=True)
    s = logits - m
    lse = jnp.log(jnp.sum(jnp.exp(s), axis=1, keepdims=True))
    out_ref[...] = s - lse


@jax.jit
def _tc_mlp(x, summed, W1, b1, W2, b2):
    return pl.pallas_call(
        _mlp_body,
        out_shape=jax.ShapeDtypeStruct((B, C), jnp.float32),
    )(x, summed, W1, b1.reshape(1, H), W2, b2.reshape(1, C))


def kernel(x, emb, W1, b1, W2, b2):
    table = _sc_fmt(emb.T).reshape(V, D)
    summed = _sc_pool(x, table)
    return _tc_mlp(x, summed, W1, b1, W2, b2)


# speedup vs baseline: 1.3603x; 1.3603x over previous
import jax
import jax.numpy as jnp
from jax import lax
from jax.experimental import pallas as pl
from jax.experimental.pallas import tpu as pltpu
from jax.experimental.pallas import tpu_sc as plsc

B = 4096
S = 200
D = 64
H = 256
C = 2

NC = 2
NS = 16
NW = NC * NS
BPW = B // NW

_S0 = 128
_S1 = S - _S0


def _sc_pool_body(x_hbm, emb_hbm, out_hbm, idx_v, rows0, rows1, out_v, sem0, sem1):
    wid = lax.axis_index("s") * NC + lax.axis_index("c")
    base = wid * BPW

    pltpu.sync_copy(x_hbm.at[pl.ds(base, BPW)], idx_v)

    def fire(b, buf, sem):
        pltpu.async_copy(emb_hbm.at[idx_v.at[b, pl.ds(0, _S0)]],
                         buf.at[pl.ds(0, _S0)], sem)
        pltpu.async_copy(emb_hbm.at[idx_v.at[b, pl.ds(_S0, _S1)]],
                         buf.at[pl.ds(_S0, _S1)], sem)

    def drain(b, buf, sem):
        pltpu.make_async_copy(emb_hbm.at[idx_v.at[b, pl.ds(0, _S0)]],
                              buf.at[pl.ds(0, _S0)], sem).wait()
        pltpu.make_async_copy(emb_hbm.at[idx_v.at[b, pl.ds(_S0, _S1)]],
                              buf.at[pl.ds(_S0, _S1)], sem).wait()

    def process(b, buf, sem):
        drain(b, buf, sem)
        zero = jnp.zeros((16,), jnp.float32)

        def acc_body(j, accs):
            return tuple(a + buf[j, pl.ds(16 * k, 16)] for k, a in enumerate(accs))

        a0, a1, a2, a3 = lax.fori_loop(0, S, acc_body, (zero, zero, zero, zero))

        out_v[b, pl.ds(0, 16)] = a0
        out_v[b, pl.ds(16, 16)] = a1
        out_v[b, pl.ds(32, 16)] = a2
        out_v[b, pl.ds(48, 16)] = a3

    fire(0, rows0, sem0)
    fire(1, rows1, sem1)

    def loop_body(g, carry):
        b = 2 * g
        process(b, rows0, sem0)

        @pl.when(b + 2 < BPW)
        def _():
            fire(b + 2, rows0, sem0)

        process(b + 1, rows1, sem1)

        @pl.when(b + 3 < BPW)
        def _():
            fire(b + 3, rows1, sem1)

        return carry

    lax.fori_loop(0, BPW // 2, loop_body, jnp.int32(0))

    pltpu.sync_copy(out_v, out_hbm.at[pl.ds(base, BPW)])


@jax.jit
def _sc_pool(x, emb):
    mesh = plsc.VectorSubcoreMesh(core_axis_name="c", subcore_axis_name="s")
    return pl.kernel(
        _sc_pool_body,
        out_type=jax.ShapeDtypeStruct((B, D), jnp.float32),
        mesh=mesh,
        scratch_types=[
            pltpu.VMEM((BPW, S), jnp.int32),
            pltpu.VMEM((S, D), jnp.float32),
            pltpu.VMEM((S, D), jnp.float32),
            pltpu.VMEM((BPW, D), jnp.float32),
            pltpu.SemaphoreType.DMA,
            pltpu.SemaphoreType.DMA,
        ],
        compiler_params=pltpu.CompilerParams(use_tc_tiling_on_sc=False),
    )(x, emb)


def _mlp_body(x_ref, summed_ref, W1_ref, b1_ref, W2_ref, b2_ref, out_ref):
    denom = jnp.sum((x_ref[...] != 0).astype(jnp.float32), axis=1, keepdims=True)
    avg = summed_ref[...] / jnp.maximum(denom, 1.0)
    h = jnp.dot(avg, W1_ref[...], preferred_element_type=jnp.float32)
    h = jnp.maximum(h + b1_ref[...], 0.0)
    logits = jnp.dot(h, W2_ref[...], preferred_element_type=jnp.float32)
    logits = logits + b2_ref[...]
    m = jnp.max(logits, axis=1, keepdims=True)
    s = logits - m
    lse = jnp.log(jnp.sum(jnp.exp(s), axis=1, keepdims=True))
    out_ref[...] = s - lse


@jax.jit
def _tc_mlp(x, summed, W1, b1, W2, b2):
    return pl.pallas_call(
        _mlp_body,
        out_shape=jax.ShapeDtypeStruct((B, C), jnp.float32),
    )(x, summed, W1, b1.reshape(1, H), W2, b2.reshape(1, C))


def kernel(x, emb, W1, b1, W2, b2):
    summed = _sc_pool(x, emb)
    return _tc_mlp(x, summed, W1, b1, W2, b2)


# speedup vs baseline: 1.4182x; 1.0426x over previous
import jax
import jax.numpy as jnp
from jax import lax
from jax.experimental import pallas as pl
from jax.experimental.pallas import tpu as pltpu
from jax.experimental.pallas import tpu_sc as plsc

B = 4096
S = 200
D = 64
H = 256
C = 2

NC = 2
NS = 16
NW = NC * NS
BPW = B // NW

_S0 = 128
_S1 = S - _S0


def _sc_pool_body(x_hbm, emb_hbm, out_hbm, idx_v, rows0, rows1, rows2, rows3,
                  out_v, sem0, sem1, sem2, sem3):
    wid = lax.axis_index("s") * NC + lax.axis_index("c")
    base = wid * BPW

    pltpu.sync_copy(x_hbm.at[pl.ds(base, BPW)], idx_v)

    def fire(b, buf, sem):
        pltpu.async_copy(emb_hbm.at[idx_v.at[b, pl.ds(0, _S0)]],
                         buf.at[pl.ds(0, _S0)], sem)
        pltpu.async_copy(emb_hbm.at[idx_v.at[b, pl.ds(_S0, _S1)]],
                         buf.at[pl.ds(_S0, _S1)], sem)

    def drain(b, buf, sem):
        pltpu.make_async_copy(emb_hbm.at[idx_v.at[b, pl.ds(0, _S0)]],
                              buf.at[pl.ds(0, _S0)], sem).wait()
        pltpu.make_async_copy(emb_hbm.at[idx_v.at[b, pl.ds(_S0, _S1)]],
                              buf.at[pl.ds(_S0, _S1)], sem).wait()

    def process(b, buf, sem):
        drain(b, buf, sem)
        zero = jnp.zeros((16,), jnp.float32)

        def acc_body(j, accs):
            return tuple(a + buf[j, pl.ds(16 * k, 16)] for k, a in enumerate(accs))

        a0, a1, a2, a3 = lax.fori_loop(0, S, acc_body, (zero, zero, zero, zero))

        out_v[b, pl.ds(0, 16)] = a0
        out_v[b, pl.ds(16, 16)] = a1
        out_v[b, pl.ds(32, 16)] = a2
        out_v[b, pl.ds(48, 16)] = a3

    bufs = ((rows0, sem0), (rows1, sem1), (rows2, sem2), (rows3, sem3))
    for p, (buf, sem) in enumerate(bufs):
        fire(p, buf, sem)

    def loop_body(g, carry):
        b = 4 * g
        for p, (buf, sem) in enumerate(bufs):
            process(b + p, buf, sem)

            @pl.when(b + p + 4 < BPW)
            def _():
                fire(b + p + 4, buf, sem)

        return carry

    lax.fori_loop(0, BPW // 4, loop_body, jnp.int32(0))

    pltpu.sync_copy(out_v, out_hbm.at[pl.ds(base, BPW)])


@jax.jit
def _sc_pool(x, emb):
    mesh = plsc.VectorSubcoreMesh(core_axis_name="c", subcore_axis_name="s")
    return pl.kernel(
        _sc_pool_body,
        out_type=jax.ShapeDtypeStruct((B, D), jnp.float32),
        mesh=mesh,
        scratch_types=[
            pltpu.VMEM((BPW, S), jnp.int32),
            pltpu.VMEM((S, D), jnp.float32),
            pltpu.VMEM((S, D), jnp.float32),
            pltpu.VMEM((S, D), jnp.float32),
            pltpu.VMEM((S, D), jnp.float32),
            pltpu.VMEM((BPW, D), jnp.float32),
            pltpu.SemaphoreType.DMA,
            pltpu.SemaphoreType.DMA,
            pltpu.SemaphoreType.DMA,
            pltpu.SemaphoreType.DMA,
        ],
        compiler_params=pltpu.CompilerParams(use_tc_tiling_on_sc=False),
    )(x, emb)


def _mlp_body(x_ref, summed_ref, W1_ref, b1_ref, W2_ref, b2_ref, out_ref):
    denom = jnp.sum((x_ref[...] != 0).astype(jnp.float32), axis=1, keepdims=True)
    avg = summed_ref[...] / jnp.maximum(denom, 1.0)
    h = jnp.dot(avg, W1_ref[...], preferred_element_type=jnp.float32)
    h = jnp.maximum(h + b1_ref[...], 0.0)
    logits = jnp.dot(h, W2_ref[...], preferred_element_type=jnp.float32)
    logits = logits + b2_ref[...]
    m = jnp.max(logits, axis=1, keepdims=True)
    s = logits - m
    lse = jnp.log(jnp.sum(jnp.exp(s), axis=1, keepdims=True))
    out_ref[...] = s - lse


@jax.jit
def _tc_mlp(x, summed, W1, b1, W2, b2):
    return pl.pallas_call(
        _mlp_body,
        out_shape=jax.ShapeDtypeStruct((B, C), jnp.float32),
    )(x, summed, W1, b1.reshape(1, H), W2, b2.reshape(1, C))


def kernel(x, emb, W1, b1, W2, b2):
    summed = _sc_pool(x, emb)
    return _tc_mlp(x, summed, W1, b1, W2, b2)
